# baseline (speedup 1.0000x reference)
import functools

import jax
import jax.numpy as jnp
from jax import lax
from jax.experimental import pallas as pl
from jax.experimental.pallas import tpu as pltpu
from jax.experimental.pallas import tpu_sc as plsc

HIDDEN = 32
NC, NS = 2, 16
NW = NC * NS
NBUF = 4
GROUP = 2
K = 2


def _emb_kernel(n_rows, n_cols):
    rows_w = n_rows // NW
    units_w = rows_w // GROUP
    nsu = units_w // K
    gcols = GROUP * n_cols
    mesh = plsc.VectorSubcoreMesh(
        core_axis_name="c", subcore_axis_name="s",
        num_cores=NC, num_subcores=NS)

    @functools.partial(
        pl.kernel,
        out_type=jax.ShapeDtypeStruct(
            (n_rows // GROUP, gcols, HIDDEN), jnp.float32),
        mesh=mesh,
        scratch_types=[
            pltpu.VMEM((units_w, gcols), jnp.int32),
            pltpu.VMEM((NBUF, K, gcols, HIDDEN), jnp.float32),
        ] + [pltpu.SemaphoreType.DMA] * (2 * NBUF),
        compiler_params=pltpu.CompilerParams(use_tc_tiling_on_sc=False),
    )
    def body(w_hbm, xg_hbm, out_hbm, idx_v, rows_v, *sems):
        gsem = sems[:NBUF]
        wsem = sems[NBUF:]
        wid = lax.axis_index("s") * NC + lax.axis_index("c")
        ubase = wid * units_w

        pltpu.sync_copy(xg_hbm.at[pl.ds(ubase, units_w)], idx_v)

        def g_start(b, su):
            for g in range(K):
                pltpu.async_copy(w_hbm.at[idx_v.at[su * K + g]],
                                 rows_v.at[b, g], gsem[b])

        def g_wait(b, su):
            for g in range(K):
                pltpu.make_async_copy(w_hbm.at[idx_v.at[su * K + g]],
                                      rows_v.at[b, g], gsem[b]).wait()

        def out_box(su):
            return out_hbm.at[pl.ds(ubase + su * K, K)]

        def w_start(b, su):
            pltpu.async_copy(rows_v.at[b], out_box(su), wsem[b])

        def w_wait(b, su):
            pltpu.make_async_copy(rows_v.at[b], out_box(su), wsem[b]).wait()

        for b in range(NBUF):
            g_start(b, b)

        @pl.loop(0, nsu, step=NBUF)
        def _(su):
            for b in range(NBUF):
                g_wait(b, su + b)
                w_start(b, su + b)
            for b in range(NBUF):
                nxt = su + b + NBUF

                @pl.when(nxt < nsu)
                def _():
                    w_wait(b, su + b)
                    g_start(b, nxt)

        for b in range(NBUF):
            w_wait(b, nsu - NBUF + b)

    return body


def kernel(x, weight):
    s0, s1 = x.shape
    xg = x.astype(jnp.int32).reshape(s0 // GROUP, GROUP * s1)
    out = _emb_kernel(s0, s1)(weight, xg)
    return out.reshape(s0, s1, HIDDEN)


# speedup vs baseline: 1.2840x; 1.2840x over previous
import functools

import jax
import jax.numpy as jnp
from jax import lax
from jax.experimental import pallas as pl
from jax.experimental.pallas import tpu as pltpu
from jax.experimental.pallas import tpu_sc as plsc

HIDDEN = 32
NC, NS = 2, 16
NW = NC * NS
NBUF = 4
GROUP = 2
K = 1


def _emb_kernel(n_rows, n_cols):
    rows_w = n_rows // NW
    units_w = rows_w // GROUP
    nsu = units_w // K
    gcols = GROUP * n_cols
    mesh = plsc.VectorSubcoreMesh(
        core_axis_name="c", subcore_axis_name="s",
        num_cores=NC, num_subcores=NS)

    @functools.partial(
        pl.kernel,
        out_type=jax.ShapeDtypeStruct((n_rows, n_cols, HIDDEN), jnp.float32),
        mesh=mesh,
        scratch_types=[
            pltpu.VMEM((units_w, gcols), jnp.int32),
            pltpu.VMEM((NBUF, K, gcols, HIDDEN), jnp.float32),
        ] + [pltpu.SemaphoreType.DMA] * (2 * NBUF),
        compiler_params=pltpu.CompilerParams(use_tc_tiling_on_sc=False),
    )
    def body(w_hbm, xg_hbm, out_hbm, idx_v, rows_v, *sems):
        gsem = sems[:NBUF]
        wsem = sems[NBUF:]
        wid = lax.axis_index("s") * NC + lax.axis_index("c")
        ubase = wid * units_w

        pltpu.sync_copy(xg_hbm.at[pl.ds(ubase, units_w)], idx_v)

        def g_start(b, su):
            for g in range(K):
                pltpu.async_copy(w_hbm.at[idx_v.at[su * K + g]],
                                 rows_v.at[b, g], gsem[b])

        def g_wait(b, su):
            for g in range(K):
                pltpu.make_async_copy(w_hbm.at[idx_v.at[su * K + g]],
                                      rows_v.at[b, g], gsem[b]).wait()

        def w_start(b, su):
            for g in range(K * GROUP):
                pltpu.async_copy(
                    rows_v.at[b, g // GROUP,
                              pl.ds((g % GROUP) * n_cols, n_cols)],
                    out_hbm.at[(ubase + su * K) * GROUP + g], wsem[b])

        def w_wait(b, su):
            for g in range(K * GROUP):
                pltpu.make_async_copy(
                    rows_v.at[b, g // GROUP,
                              pl.ds((g % GROUP) * n_cols, n_cols)],
                    out_hbm.at[(ubase + su * K) * GROUP + g], wsem[b]).wait()

        for b in range(NBUF):
            g_start(b, b)

        @pl.loop(0, nsu, step=NBUF)
        def _(su):
            for b in range(NBUF):
                g_wait(b, su + b)
                w_start(b, su + b)
            for b in range(NBUF):
                nxt = su + b + NBUF

                @pl.when(nxt < nsu)
                def _():
                    w_wait(b, su + b)
                    g_start(b, nxt)

        for b in range(NBUF):
            w_wait(b, nsu - NBUF + b)

    return body


def kernel(x, weight):
    s0, s1 = x.shape
    xg = x.astype(jnp.int32).reshape(s0 // GROUP, GROUP * s1)
    out = _emb_kernel(s0, s1)(weight, xg)
    return out.reshape(s0, s1, HIDDEN)


# speedup vs baseline: 1.2853x; 1.0010x over previous
import functools

import jax
import jax.numpy as jnp
from jax import lax
from jax.experimental import pallas as pl
from jax.experimental.pallas import tpu as pltpu
from jax.experimental.pallas import tpu_sc as plsc

HIDDEN = 32
NC, NS = 2, 16
NW = NC * NS
NBUF = 4
GROUP = 2


def _emb_kernel(n_rows, n_cols):
    rows_w = n_rows // NW
    units_w = rows_w // GROUP
    gcols = GROUP * n_cols
    mesh = plsc.VectorSubcoreMesh(
        core_axis_name="c", subcore_axis_name="s",
        num_cores=NC, num_subcores=NS)

    @functools.partial(
        pl.kernel,
        out_type=jax.ShapeDtypeStruct((n_rows, n_cols, HIDDEN), jnp.float32),
        mesh=mesh,
        scratch_types=[
            pltpu.VMEM((units_w, gcols), jnp.int32),
            pltpu.VMEM((NBUF, gcols, HIDDEN), jnp.float32),
        ] + [pltpu.SemaphoreType.DMA] * (2 * NBUF),
        compiler_params=pltpu.CompilerParams(use_tc_tiling_on_sc=False),
    )
    def body(w_hbm, xg_hbm, out_hbm, idx_v, rows_v, *sems):
        gsem = sems[:NBUF]
        wsem = sems[NBUF:]
        wid = lax.axis_index("s") * NC + lax.axis_index("c")
        ubase = wid * units_w

        pltpu.sync_copy(xg_hbm.at[pl.ds(ubase, units_w)], idx_v)

        def g_start(b, u):
            pltpu.async_copy(w_hbm.at[idx_v.at[u]], rows_v.at[b], gsem[b])

        def g_wait(b, u):
            pltpu.make_async_copy(
                w_hbm.at[idx_v.at[u]], rows_v.at[b], gsem[b]).wait()

        def w_start(b, u):
            for g in range(GROUP):
                pltpu.async_copy(
                    rows_v.at[b, pl.ds(g * n_cols, n_cols)],
                    out_hbm.at[(ubase + u) * GROUP + g], wsem[b])

        def w_wait(b, u):
            for g in range(GROUP):
                pltpu.make_async_copy(
                    rows_v.at[b, pl.ds(g * n_cols, n_cols)],
                    out_hbm.at[(ubase + u) * GROUP + g], wsem[b]).wait()

        for b in range(NBUF):
            g_start(b, b)

        @pl.loop(0, units_w, step=NBUF)
        def _(u):
            for b in range(NBUF):
                g_wait(b, u + b)
                w_start(b, u + b)
            for b in range(NBUF):
                nxt = u + b + NBUF

                @pl.when(nxt < units_w)
                def _():
                    w_wait(b, u + b)
                    g_start(b, nxt)

        for b in range(NBUF):
            w_wait(b, units_w - NBUF + b)

    return body


def kernel(x, weight):
    s0, s1 = x.shape
    xg = x.astype(jnp.int32).reshape(s0 // GROUP, GROUP * s1)
    out = _emb_kernel(s0, s1)(weight, xg)
    return out.reshape(s0, s1, HIDDEN)
